# baseline (speedup 1.0000x reference)
import functools

import jax
import jax.numpy as jnp
from jax import lax
from jax.experimental import pallas as pl
from jax.experimental.pallas import tpu as pltpu
from jax.experimental.pallas import tpu_sc as plsc

T = 4096
D = 1024
H = 512
E = 16
K = 2

BLK = 128
P = T * K + E * BLK
NB = P // BLK

NC = 2
NS = 16
NW = NC * NS
TW = T // NW
CHD = 32
CHC = 16


def _router_body(x_ref, gt_ref, pos0_ref, pos1_ref, wts_ref, be_ref):
    x = x_ref[...]
    logits = jnp.dot(x, gt_ref[...], preferred_element_type=jnp.float32)
    m = jnp.max(logits, axis=-1, keepdims=True)
    ex = jnp.exp(logits - m)
    scores = ex / jnp.sum(ex, axis=-1, keepdims=True)

    eiota = lax.broadcasted_iota(jnp.int32, (T, E), 1)
    m1 = jnp.max(scores, axis=-1, keepdims=True)
    i1 = jnp.min(jnp.where(scores == m1, eiota, E), axis=-1, keepdims=True)
    masked = jnp.where(eiota == i1, -jnp.inf, scores)
    m2 = jnp.max(masked, axis=-1, keepdims=True)
    i2 = jnp.min(jnp.where(masked == m2, eiota, E), axis=-1, keepdims=True)

    oh1 = (i1 == eiota).astype(jnp.float32)
    oh2 = (i2 == eiota).astype(jnp.float32)
    cnt1 = jnp.sum(oh1, axis=0, keepdims=True)
    cnt2 = jnp.sum(oh2, axis=0, keepdims=True)
    cnt = cnt1 + cnt2

    pc = jnp.ceil(cnt * (1.0 / BLK)) * BLK
    li = lax.broadcasted_iota(jnp.int32, (E, E), 0)
    lj = lax.broadcasted_iota(jnp.int32, (E, E), 1)
    lmat = (li < lj).astype(jnp.float32)
    starts = jnp.dot(pc, lmat, preferred_element_type=jnp.float32)
    ends = starts + pc

    C = 512
    ri = lax.broadcasted_iota(jnp.int32, (C, C), 0)
    rj = lax.broadcasted_iota(jnp.int32, (C, C), 1)
    tri = (rj < ri).astype(jnp.float32)
    oh = jnp.concatenate([oh1, oh2], axis=1)
    carry = jnp.concatenate([jnp.zeros((1, E), jnp.float32), cnt1], axis=1)
    r1p, r2p = [], []
    for c in range(T // C):
        b = oh[c * C:(c + 1) * C]
        eb = jnp.dot(tri, b, preferred_element_type=jnp.float32) + carry
        rb = b * eb
        r1p.append(jnp.sum(rb[:, :E], axis=1, keepdims=True))
        r2p.append(jnp.sum(rb[:, E:], axis=1, keepdims=True))
        carry = carry + jnp.sum(b, axis=0, keepdims=True)
    r1 = jnp.concatenate(r1p, axis=0)
    r2 = jnp.concatenate(r2p, axis=0)

    s1 = jnp.sum(oh1 * starts, axis=1, keepdims=True)
    s2 = jnp.sum(oh2 * starts, axis=1, keepdims=True)
    pos0_ref[...] = (s1 + r1).astype(jnp.int32)
    pos1_ref[...] = (s2 + r2).astype(jnp.int32)

    wts_ref[...] = jnp.concatenate(
        [jnp.broadcast_to(m1, (T, 16)), jnp.broadcast_to(m2, (T, 16))], axis=1)

    bstart = (lax.broadcasted_iota(jnp.int32, (128, 1), 0) * BLK
              ).astype(jnp.float32)
    be_ref[...] = jnp.sum((ends <= bstart).astype(jnp.int32), axis=1,
                          keepdims=True)


def _router(x, gate_t):
    return pl.pallas_call(
        _router_body,
        grid=(1,),
        in_specs=[
            pl.BlockSpec((T, D), lambda i: (0, 0)),
            pl.BlockSpec((D, E), lambda i: (0, 0)),
        ],
        out_specs=[
            pl.BlockSpec((T, 1), lambda i: (0, 0)),
            pl.BlockSpec((T, 1), lambda i: (0, 0)),
            pl.BlockSpec((T, K * 16), lambda i: (0, 0)),
            pl.BlockSpec((128, 1), lambda i: (0, 0)),
        ],
        out_shape=[
            jax.ShapeDtypeStruct((T, 1), jnp.int32),
            jax.ShapeDtypeStruct((T, 1), jnp.int32),
            jax.ShapeDtypeStruct((T, K * 16), jnp.float32),
            jax.ShapeDtypeStruct((128, 1), jnp.int32),
        ],
    )(x, gate_t)


@functools.cache
def _sc_mesh():
    return plsc.VectorSubcoreMesh(core_axis_name="c", subcore_axis_name="s",
                                  num_cores=NC, num_subcores=NS)


def _wid():
    return lax.axis_index("s") * NC + lax.axis_index("c")


def _dispatch_body(x_hbm, p0_hbm, p1_hbm, xs_hbm, i0b, i1b, xb, lsem, ssem):
    wid = _wid()
    base = wid * TW
    nch = TW // CHD
    pltpu.sync_copy(p0_hbm.at[wid], i0b)
    pltpu.sync_copy(p1_hbm.at[wid], i1b)
    lcp = [None] * nch
    s0 = [None] * nch
    s1 = [None] * nch

    def load(c):
        lcp[c] = pltpu.async_copy(
            x_hbm.at[pl.ds(base + c * CHD, CHD)], xb.at[c % 2], lsem)

    load(0)
    for c in range(nch):
        b = c % 2
        if c + 1 < nch:
            if c - 1 >= 0:
                s0[c - 1].wait()
                s1[c - 1].wait()
            load(c + 1)
        lcp[c].wait()
        s0[c] = pltpu.async_copy(xb.at[b], xs_hbm.at[i0b.at[c]], ssem)
        s1[c] = pltpu.async_copy(xb.at[b], xs_hbm.at[i1b.at[c]], ssem)
    for c in range(max(0, nch - 2), nch):
        s0[c].wait()
        s1[c].wait()


@functools.cache
def _dispatch():
    return pl.kernel(
        _dispatch_body,
        out_type=jax.ShapeDtypeStruct((P, D), jnp.float32),
        mesh=_sc_mesh(),
        scratch_types=[
            pltpu.VMEM((TW // CHD, CHD), jnp.int32),
            pltpu.VMEM((TW // CHD, CHD), jnp.int32),
            pltpu.VMEM((2, CHD, D), jnp.float32),
            pltpu.SemaphoreType.DMA,
            pltpu.SemaphoreType.DMA,
        ],
    )


def _gmm_body(be_ref, xs_ref, w1_ref, w3_ref, w2_ref, ys_ref,
              w1s_ref, w3s_ref, w2s_ref):
    b = pl.program_id(0)
    be = be_ref[b]
    prev = be_ref[jnp.maximum(b - 1, 0)]
    live = be < E

    @pl.when(live & ((b == 0) | (be != prev)))
    def _():
        w1s_ref[...] = w1_ref[0].astype(jnp.bfloat16)
        w3s_ref[...] = w3_ref[0].astype(jnp.bfloat16)
        w2s_ref[...] = w2_ref[0].astype(jnp.bfloat16)

    @pl.when(live)
    def _():
        x = xs_ref[...].astype(jnp.bfloat16)
        a = lax.dot_general(x, w1s_ref[...], (((1,), (1,)), ((), ())),
                            preferred_element_type=jnp.float32)
        g = lax.dot_general(x, w3s_ref[...], (((1,), (1,)), ((), ())),
                            preferred_element_type=jnp.float32)
        h = ((a * jax.nn.sigmoid(a)) * g).astype(jnp.bfloat16)
        ys_ref[...] = lax.dot_general(h, w2s_ref[...], (((1,), (1,)), ((), ())),
                                      preferred_element_type=jnp.float32)


def _gmm(be_flat, xs, w1b, w3b, w2b):
    def wsel(b, be):
        return (jnp.minimum(be[b], E - 1), 0, 0)

    grid_spec = pltpu.PrefetchScalarGridSpec(
        num_scalar_prefetch=1,
        grid=(NB,),
        in_specs=[
            pl.BlockSpec((BLK, D), lambda b, be: (b, 0)),
            pl.BlockSpec((1, H, D), wsel),
            pl.BlockSpec((1, H, D), wsel),
            pl.BlockSpec((1, D, H), wsel),
        ],
        out_specs=pl.BlockSpec((BLK, D), lambda b, be: (b, 0)),
        scratch_shapes=[
            pltpu.VMEM((H, D), jnp.bfloat16),
            pltpu.VMEM((H, D), jnp.bfloat16),
            pltpu.VMEM((D, H), jnp.bfloat16),
        ],
    )
    return pl.pallas_call(
        _gmm_body,
        grid_spec=grid_spec,
        out_shape=jax.ShapeDtypeStruct((P, D), jnp.float32),
    )(be_flat, xs, w1b, w3b, w2b)


def _combine_body(ys_hbm, p0_hbm, p1_hbm, wt_hbm, out_hbm,
                  i0b, i1b, wb, y0, y1, gsem, osem):
    wid = _wid()
    base = wid * TW
    nch = TW // CHC
    pltpu.sync_copy(p0_hbm.at[wid], i0b)
    pltpu.sync_copy(p1_hbm.at[wid], i1b)
    pltpu.sync_copy(wt_hbm.at[wid], wb)
    g0 = [None] * nch
    g1 = [None] * nch
    ow = [None] * nch

    def issue(c):
        b = c % 3
        g0[c] = pltpu.async_copy(ys_hbm.at[i0b.at[c]], y0.at[b], gsem)
        g1[c] = pltpu.async_copy(ys_hbm.at[i1b.at[c]], y1.at[b], gsem)

    issue(0)
    issue(1)
    for c in range(nch):
        b = c % 3
        if c + 2 < nch:
            if c - 1 >= 0:
                ow[c - 1].wait()
            issue(c + 2)
        g0[c].wait()
        g1[c].wait()

        def tok(j, _):
            w0 = wb[c, j, pl.ds(0, 16)]
            w1v = wb[c, j, pl.ds(16, 16)]
            for v in range(D // 16):
                sl = pl.ds(v * 16, 16)
                y0[b, j, sl] = w0 * y0[b, j, sl] + w1v * y1[b, j, sl]
            return 0

        lax.fori_loop(0, CHC, tok, 0)
        ow[c] = pltpu.async_copy(
            y0.at[b], out_hbm.at[pl.ds(base + c * CHC, CHC)], osem)
    for c in range(max(0, nch - 3), nch):
        ow[c].wait()


@functools.cache
def _combine():
    return pl.kernel(
        _combine_body,
        out_type=jax.ShapeDtypeStruct((T, D), jnp.float32),
        mesh=_sc_mesh(),
        scratch_types=[
            pltpu.VMEM((TW // CHC, CHC), jnp.int32),
            pltpu.VMEM((TW // CHC, CHC), jnp.int32),
            pltpu.VMEM((TW // CHC, CHC, K * 16), jnp.float32),
            pltpu.VMEM((3, CHC, D), jnp.float32),
            pltpu.VMEM((3, CHC, D), jnp.float32),
            pltpu.SemaphoreType.DMA,
            pltpu.SemaphoreType.DMA,
        ],
    )


@jax.jit
def kernel(x, gate, w1, w2, w3):
    pos0, pos1, wts, be = _router(x, gate.T)
    be_flat = be.reshape(128)

    p0d = pos0.reshape(NW, TW // CHD, CHD)
    p1d = pos1.reshape(NW, TW // CHD, CHD)
    p0c = pos0.reshape(NW, TW // CHC, CHC)
    p1c = pos1.reshape(NW, TW // CHC, CHC)
    wtc = wts.reshape(NW, TW // CHC, CHC, K * 16)

    xs = _dispatch()(x, p0d, p1d)
    ys = _gmm(be_flat, xs, w1, w3, w2)
    return _combine()(ys, p0c, p1c, wtc)


# speedup vs baseline: 1.2655x; 1.2655x over previous
import functools

import jax
import jax.numpy as jnp
from jax import lax
from jax.experimental import pallas as pl
from jax.experimental.pallas import tpu as pltpu
from jax.experimental.pallas import tpu_sc as plsc

T = 4096
D = 1024
H = 512
E = 16
K = 2

BLK = 256
P = T * K + E * BLK
NB = P // BLK

NC = 2
NS = 16
NW = NC * NS
TW = T // NW
CHD = 32
CHC = 16


def _router_body(x_ref, gt_ref, pos0_ref, pos1_ref, wts_ref, be_ref):
    x = x_ref[...]
    logits = jnp.dot(x, gt_ref[...], preferred_element_type=jnp.float32)
    m = jnp.max(logits, axis=-1, keepdims=True)
    ex = jnp.exp(logits - m)
    scores = ex / jnp.sum(ex, axis=-1, keepdims=True)

    eiota = lax.broadcasted_iota(jnp.int32, (T, E), 1)
    m1 = jnp.max(scores, axis=-1, keepdims=True)
    i1 = jnp.min(jnp.where(scores == m1, eiota, E), axis=-1, keepdims=True)
    masked = jnp.where(eiota == i1, -jnp.inf, scores)
    m2 = jnp.max(masked, axis=-1, keepdims=True)
    i2 = jnp.min(jnp.where(masked == m2, eiota, E), axis=-1, keepdims=True)

    oh1 = (i1 == eiota).astype(jnp.float32)
    oh2 = (i2 == eiota).astype(jnp.float32)
    cnt1 = jnp.sum(oh1, axis=0, keepdims=True)
    cnt2 = jnp.sum(oh2, axis=0, keepdims=True)
    cnt = cnt1 + cnt2

    pc = jnp.ceil(cnt * (1.0 / BLK)) * BLK
    li = lax.broadcasted_iota(jnp.int32, (E, E), 0)
    lj = lax.broadcasted_iota(jnp.int32, (E, E), 1)
    lmat = (li < lj).astype(jnp.float32)
    starts = jnp.dot(pc, lmat, preferred_element_type=jnp.float32)
    ends = starts + pc

    C = 512
    ri = lax.broadcasted_iota(jnp.int32, (C, C), 0)
    rj = lax.broadcasted_iota(jnp.int32, (C, C), 1)
    tri = (rj < ri).astype(jnp.float32)
    oh = jnp.concatenate([oh1, oh2], axis=1)
    carry = jnp.concatenate([jnp.zeros((1, E), jnp.float32), cnt1], axis=1)
    r1p, r2p = [], []
    for c in range(T // C):
        b = oh[c * C:(c + 1) * C]
        eb = jnp.dot(tri, b, preferred_element_type=jnp.float32) + carry
        rb = b * eb
        r1p.append(jnp.sum(rb[:, :E], axis=1, keepdims=True))
        r2p.append(jnp.sum(rb[:, E:], axis=1, keepdims=True))
        carry = carry + jnp.sum(b, axis=0, keepdims=True)
    r1 = jnp.concatenate(r1p, axis=0)
    r2 = jnp.concatenate(r2p, axis=0)

    s1 = jnp.sum(oh1 * starts, axis=1, keepdims=True)
    s2 = jnp.sum(oh2 * starts, axis=1, keepdims=True)
    pos0_ref[...] = (s1 + r1).astype(jnp.int32)
    pos1_ref[...] = (s2 + r2).astype(jnp.int32)

    wts_ref[...] = jnp.concatenate(
        [jnp.broadcast_to(m1, (T, 16)), jnp.broadcast_to(m2, (T, 16))], axis=1)

    bstart = (lax.broadcasted_iota(jnp.int32, (128, 1), 0) * BLK
              ).astype(jnp.float32)
    be_ref[...] = jnp.sum((ends <= bstart).astype(jnp.int32), axis=1,
                          keepdims=True)


def _router(x, gate_t):
    return pl.pallas_call(
        _router_body,
        grid=(1,),
        in_specs=[
            pl.BlockSpec((T, D), lambda i: (0, 0)),
            pl.BlockSpec((D, E), lambda i: (0, 0)),
        ],
        out_specs=[
            pl.BlockSpec((T, 1), lambda i: (0, 0)),
            pl.BlockSpec((T, 1), lambda i: (0, 0)),
            pl.BlockSpec((T, K * 16), lambda i: (0, 0)),
            pl.BlockSpec((128, 1), lambda i: (0, 0)),
        ],
        out_shape=[
            jax.ShapeDtypeStruct((T, 1), jnp.int32),
            jax.ShapeDtypeStruct((T, 1), jnp.int32),
            jax.ShapeDtypeStruct((T, K * 16), jnp.float32),
            jax.ShapeDtypeStruct((128, 1), jnp.int32),
        ],
    )(x, gate_t)


@functools.cache
def _sc_mesh():
    return plsc.VectorSubcoreMesh(core_axis_name="c", subcore_axis_name="s",
                                  num_cores=NC, num_subcores=NS)


def _wid():
    return lax.axis_index("s") * NC + lax.axis_index("c")


def _dispatch_body(x_hbm, p0_hbm, p1_hbm, xs_hbm, i0b, i1b, xb, lsem, ssem):
    wid = _wid()
    base = wid * TW
    nch = TW // CHD
    pltpu.sync_copy(p0_hbm.at[wid], i0b)
    pltpu.sync_copy(p1_hbm.at[wid], i1b)
    lcp = [None] * nch
    s0 = [None] * nch
    s1 = [None] * nch

    def load(c):
        lcp[c] = pltpu.async_copy(
            x_hbm.at[pl.ds(base + c * CHD, CHD)], xb.at[c % 2], lsem)

    load(0)
    for c in range(nch):
        b = c % 2
        if c + 1 < nch:
            if c - 1 >= 0:
                s0[c - 1].wait()
                s1[c - 1].wait()
            load(c + 1)
        lcp[c].wait()
        s0[c] = pltpu.async_copy(xb.at[b], xs_hbm.at[i0b.at[c]], ssem)
        s1[c] = pltpu.async_copy(xb.at[b], xs_hbm.at[i1b.at[c]], ssem)
    for c in range(max(0, nch - 2), nch):
        s0[c].wait()
        s1[c].wait()


@functools.cache
def _dispatch():
    return pl.kernel(
        _dispatch_body,
        out_type=jax.ShapeDtypeStruct((P, D), jnp.float32),
        mesh=_sc_mesh(),
        scratch_types=[
            pltpu.VMEM((TW // CHD, CHD), jnp.int32),
            pltpu.VMEM((TW // CHD, CHD), jnp.int32),
            pltpu.VMEM((2, CHD, D), jnp.float32),
            pltpu.SemaphoreType.DMA,
            pltpu.SemaphoreType.DMA,
        ],
    )


def _gmm_body(be_ref, xs_ref, w1_ref, w3_ref, w2_ref, ys_ref):
    b = pl.program_id(0)

    @pl.when(be_ref[b] < E)
    def _():
        x = xs_ref[...]
        a = lax.dot_general(x, w1_ref[0], (((1,), (1,)), ((), ())),
                            preferred_element_type=jnp.float32)
        g = lax.dot_general(x, w3_ref[0], (((1,), (1,)), ((), ())),
                            preferred_element_type=jnp.float32)
        h = (a * jax.nn.sigmoid(a)) * g
        ys_ref[...] = lax.dot_general(h, w2_ref[0], (((1,), (1,)), ((), ())),
                                      preferred_element_type=jnp.float32)


def _gmm(be_flat, xs, w1b, w3b, w2b):
    def wsel(b, be):
        return (jnp.minimum(be[b], E - 1), 0, 0)

    def bsel(b, be):
        return (jnp.where(be[b] < E, b, NB - 1), 0)

    grid_spec = pltpu.PrefetchScalarGridSpec(
        num_scalar_prefetch=1,
        grid=(NB,),
        in_specs=[
            pl.BlockSpec((BLK, D), bsel),
            pl.BlockSpec((1, H, D), wsel),
            pl.BlockSpec((1, H, D), wsel),
            pl.BlockSpec((1, D, H), wsel),
        ],
        out_specs=pl.BlockSpec((BLK, D), bsel),
    )
    return pl.pallas_call(
        _gmm_body,
        grid_spec=grid_spec,
        out_shape=jax.ShapeDtypeStruct((P, D), jnp.float32),
    )(be_flat, xs, w1b, w3b, w2b)


def _combine_body(ys_hbm, p0_hbm, p1_hbm, wt_hbm, out_hbm,
                  i0b, i1b, wb, y0, y1, ob, gsem, osem):
    wid = _wid()
    base = wid * TW
    nch = TW // CHC
    pltpu.sync_copy(p0_hbm.at[wid], i0b)
    pltpu.sync_copy(p1_hbm.at[wid], i1b)
    pltpu.sync_copy(wt_hbm.at[wid], wb)
    g0 = [None] * nch
    g1 = [None] * nch
    ow = [None] * nch

    def issue(c):
        b = c % 2
        g0[c] = pltpu.async_copy(ys_hbm.at[i0b.at[c]], y0.at[b], gsem)
        g1[c] = pltpu.async_copy(ys_hbm.at[i1b.at[c]], y1.at[b], gsem)

    issue(0)
    for c in range(nch):
        b = c % 2
        if c + 1 < nch:
            issue(c + 1)
        g0[c].wait()
        g1[c].wait()
        if c - 2 >= 0:
            ow[c - 2].wait()

        def tok(j, _):
            w0 = wb[c, j, pl.ds(0, 16)]
            w1v = wb[c, j, pl.ds(16, 16)]
            for v in range(D // 16):
                sl = pl.ds(v * 16, 16)
                ob[b, j, sl] = w0 * y0[b, j, sl] + w1v * y1[b, j, sl]
            return 0

        lax.fori_loop(0, CHC, tok, 0)
        ow[c] = pltpu.async_copy(
            ob.at[b], out_hbm.at[pl.ds(base + c * CHC, CHC)], osem)
    for c in range(max(0, nch - 2), nch):
        ow[c].wait()


@functools.cache
def _combine():
    return pl.kernel(
        _combine_body,
        out_type=jax.ShapeDtypeStruct((T, D), jnp.float32),
        mesh=_sc_mesh(),
        scratch_types=[
            pltpu.VMEM((TW // CHC, CHC), jnp.int32),
            pltpu.VMEM((TW // CHC, CHC), jnp.int32),
            pltpu.VMEM((TW // CHC, CHC, K * 16), jnp.float32),
            pltpu.VMEM((2, CHC, D), jnp.float32),
            pltpu.VMEM((2, CHC, D), jnp.float32),
            pltpu.VMEM((2, CHC, D), jnp.float32),
            pltpu.SemaphoreType.DMA,
            pltpu.SemaphoreType.DMA,
        ],
    )


@jax.jit
def kernel(x, gate, w1, w2, w3):
    pos0, pos1, wts, be = _router(x, gate.T)
    be_flat = be.reshape(128)

    p0d = pos0.reshape(NW, TW // CHD, CHD)
    p1d = pos1.reshape(NW, TW // CHD, CHD)
    p0c = pos0.reshape(NW, TW // CHC, CHC)
    p1c = pos1.reshape(NW, TW // CHC, CHC)
    wtc = wts.reshape(NW, TW // CHC, CHC, K * 16)

    xs = _dispatch()(x, p0d, p1d)
    ys = _gmm(be_flat, xs, w1, w3, w2)
    return _combine()(ys, p0c, p1c, wtc)


# speedup vs baseline: 1.2940x; 1.0225x over previous
import functools

import jax
import jax.numpy as jnp
from jax import lax
from jax.experimental import pallas as pl
from jax.experimental.pallas import tpu as pltpu
from jax.experimental.pallas import tpu_sc as plsc

T = 4096
D = 1024
H = 512
E = 16
K = 2

BLK = 256
P = T * K + E * BLK
NB = P // BLK

NC = 2
NS = 16
NW = NC * NS
TW = T // NW
CHD = 32
CHC = 16


def _router_body(x_ref, gt_ref, pos0_ref, pos1_ref, wts_ref, be_ref):
    x = x_ref[...]
    logits = lax.dot_general(x, gt_ref[...], (((1,), (1,)), ((), ())),
                             preferred_element_type=jnp.float32)
    m = jnp.max(logits, axis=-1, keepdims=True)
    ex = jnp.exp(logits - m)
    scores = ex / jnp.sum(ex, axis=-1, keepdims=True)

    eiota = lax.broadcasted_iota(jnp.int32, (T, E), 1)
    m1 = jnp.max(scores, axis=-1, keepdims=True)
    i1 = jnp.min(jnp.where(scores == m1, eiota, E), axis=-1, keepdims=True)
    masked = jnp.where(eiota == i1, -jnp.inf, scores)
    m2 = jnp.max(masked, axis=-1, keepdims=True)
    i2 = jnp.min(jnp.where(masked == m2, eiota, E), axis=-1, keepdims=True)

    oh1 = (i1 == eiota).astype(jnp.float32)
    oh2 = (i2 == eiota).astype(jnp.float32)
    cnt1 = jnp.sum(oh1, axis=0, keepdims=True)
    cnt2 = jnp.sum(oh2, axis=0, keepdims=True)
    cnt = cnt1 + cnt2

    pc = jnp.ceil(cnt * (1.0 / BLK)) * BLK
    li = lax.broadcasted_iota(jnp.int32, (E, E), 0)
    lj = lax.broadcasted_iota(jnp.int32, (E, E), 1)
    lmat = (li < lj).astype(jnp.float32)
    starts = jnp.dot(pc, lmat, preferred_element_type=jnp.float32)
    ends = starts + pc

    C = 512
    ri = lax.broadcasted_iota(jnp.int32, (C, C), 0)
    rj = lax.broadcasted_iota(jnp.int32, (C, C), 1)
    tri = (rj < ri).astype(jnp.float32)
    oh = jnp.concatenate([oh1, oh2], axis=1)
    carry = jnp.concatenate([jnp.zeros((1, E), jnp.float32), cnt1], axis=1)
    r1p, r2p = [], []
    for c in range(T // C):
        b = oh[c * C:(c + 1) * C]
        eb = jnp.dot(tri, b, preferred_element_type=jnp.float32) + carry
        rb = b * eb
        r1p.append(jnp.sum(rb[:, :E], axis=1, keepdims=True))
        r2p.append(jnp.sum(rb[:, E:], axis=1, keepdims=True))
        carry = carry + jnp.sum(b, axis=0, keepdims=True)
    r1 = jnp.concatenate(r1p, axis=0)
    r2 = jnp.concatenate(r2p, axis=0)

    s1 = jnp.sum(oh1 * starts, axis=1, keepdims=True)
    s2 = jnp.sum(oh2 * starts, axis=1, keepdims=True)
    pos0_ref[...] = (s1 + r1).astype(jnp.int32)
    pos1_ref[...] = (s2 + r2).astype(jnp.int32)

    wts_ref[...] = jnp.concatenate(
        [jnp.broadcast_to(m1, (T, 16)), jnp.broadcast_to(m2, (T, 16))], axis=1)

    bstart = (lax.broadcasted_iota(jnp.int32, (128, 1), 0) * BLK
              ).astype(jnp.float32)
    be_ref[...] = jnp.sum((ends <= bstart).astype(jnp.int32), axis=1,
                          keepdims=True)


def _router(x, gate):
    return pl.pallas_call(
        _router_body,
        grid=(1,),
        in_specs=[
            pl.BlockSpec((T, D), lambda i: (0, 0)),
            pl.BlockSpec((E, D), lambda i: (0, 0)),
        ],
        out_specs=[
            pl.BlockSpec((T, 1), lambda i: (0, 0)),
            pl.BlockSpec((T, 1), lambda i: (0, 0)),
            pl.BlockSpec((T, K * 16), lambda i: (0, 0)),
            pl.BlockSpec((128, 1), lambda i: (0, 0)),
        ],
        out_shape=[
            jax.ShapeDtypeStruct((T, 1), jnp.int32),
            jax.ShapeDtypeStruct((T, 1), jnp.int32),
            jax.ShapeDtypeStruct((T, K * 16), jnp.float32),
            jax.ShapeDtypeStruct((128, 1), jnp.int32),
        ],
    )(x, gate)


@functools.cache
def _sc_mesh():
    return plsc.VectorSubcoreMesh(core_axis_name="c", subcore_axis_name="s",
                                  num_cores=NC, num_subcores=NS)


def _wid():
    return lax.axis_index("s") * NC + lax.axis_index("c")


def _dispatch_body(x_hbm, p0_hbm, p1_hbm, xs_hbm, i0b, i1b, xb, lsem, ssem):
    wid = _wid()
    base = wid * TW
    nch = TW // CHD
    pltpu.sync_copy(p0_hbm.at[wid], i0b)
    pltpu.sync_copy(p1_hbm.at[wid], i1b)
    lcp = [None] * nch
    s0 = [None] * nch
    s1 = [None] * nch

    def load(c):
        lcp[c] = pltpu.async_copy(
            x_hbm.at[pl.ds(base + c * CHD, CHD)], xb.at[c % 2], lsem)

    load(0)
    for c in range(nch):
        b = c % 2
        if c + 1 < nch:
            if c - 1 >= 0:
                s0[c - 1].wait()
                s1[c - 1].wait()
            load(c + 1)
        lcp[c].wait()
        s0[c] = pltpu.async_copy(xb.at[b], xs_hbm.at[i0b.at[c]], ssem)
        s1[c] = pltpu.async_copy(xb.at[b], xs_hbm.at[i1b.at[c]], ssem)
    for c in range(max(0, nch - 2), nch):
        s0[c].wait()
        s1[c].wait()


@functools.cache
def _dispatch():
    return pl.kernel(
        _dispatch_body,
        out_type=jax.ShapeDtypeStruct((P, D), jnp.float32),
        mesh=_sc_mesh(),
        scratch_types=[
            pltpu.VMEM((TW // CHD, CHD), jnp.int32),
            pltpu.VMEM((TW // CHD, CHD), jnp.int32),
            pltpu.VMEM((2, CHD, D), jnp.float32),
            pltpu.SemaphoreType.DMA,
            pltpu.SemaphoreType.DMA,
        ],
    )


def _gmm_body(be_ref, xs_ref, w1_ref, w3_ref, w2_ref, ys_ref):
    b = pl.program_id(0)

    @pl.when(be_ref[b] < E)
    def _():
        x = xs_ref[...]
        a = lax.dot_general(x, w1_ref[0], (((1,), (1,)), ((), ())),
                            preferred_element_type=jnp.float32)
        g = lax.dot_general(x, w3_ref[0], (((1,), (1,)), ((), ())),
                            preferred_element_type=jnp.float32)
        h = (a * jax.nn.sigmoid(a)) * g
        ys_ref[...] = lax.dot_general(h, w2_ref[0], (((1,), (1,)), ((), ())),
                                      preferred_element_type=jnp.float32)


def _gmm(be_flat, xs, w1b, w3b, w2b):
    def wsel(b, be):
        return (jnp.minimum(be[b], E - 1), 0, 0)

    def bsel(b, be):
        return (jnp.where(be[b] < E, b, NB - 1), 0)

    grid_spec = pltpu.PrefetchScalarGridSpec(
        num_scalar_prefetch=1,
        grid=(NB,),
        in_specs=[
            pl.BlockSpec((BLK, D), bsel),
            pl.BlockSpec((1, H, D), wsel),
            pl.BlockSpec((1, H, D), wsel),
            pl.BlockSpec((1, D, H), wsel),
        ],
        out_specs=pl.BlockSpec((BLK, D), bsel),
    )
    return pl.pallas_call(
        _gmm_body,
        grid_spec=grid_spec,
        out_shape=jax.ShapeDtypeStruct((P, D), jnp.float32),
    )(be_flat, xs, w1b, w3b, w2b)


def _combine_body(ys_hbm, p0_hbm, p1_hbm, wt_hbm, out_hbm,
                  i0b, i1b, i01, wb, y01, ob, gsem, osem):
    wid = _wid()
    base = wid * TW
    nch = TW // CHC
    pltpu.sync_copy(p0_hbm.at[wid], i0b)
    pltpu.sync_copy(p1_hbm.at[wid], i1b)
    pltpu.sync_copy(wt_hbm.at[wid], wb)
    for c in range(nch):
        for v in range(CHC // 16):
            i01[c, pl.ds(v * 16, 16)] = i0b[c, pl.ds(v * 16, 16)]
            i01[c, pl.ds(CHC + v * 16, 16)] = i1b[c, pl.ds(v * 16, 16)]
    g = [None] * nch
    ow = [None] * nch

    def issue(c):
        g[c] = pltpu.async_copy(ys_hbm.at[i01.at[c]], y01.at[c % 2], gsem)

    issue(0)
    for c in range(nch):
        b = c % 2
        if c + 1 < nch:
            issue(c + 1)
        g[c].wait()
        if c - 2 >= 0:
            ow[c - 2].wait()

        def tok(j, _):
            w0 = wb[c, j, pl.ds(0, 16)]
            w1v = wb[c, j, pl.ds(16, 16)]
            for v in range(D // 16):
                sl = pl.ds(v * 16, 16)
                ob[b, j, sl] = (w0 * y01[b, j, sl]
                                + w1v * y01[b, CHC + j, sl])
            return 0

        lax.fori_loop(0, CHC, tok, 0)
        ow[c] = pltpu.async_copy(
            ob.at[b], out_hbm.at[pl.ds(base + c * CHC, CHC)], osem)
    for c in range(max(0, nch - 2), nch):
        ow[c].wait()


@functools.cache
def _combine():
    return pl.kernel(
        _combine_body,
        out_type=jax.ShapeDtypeStruct((T, D), jnp.float32),
        mesh=_sc_mesh(),
        scratch_types=[
            pltpu.VMEM((TW // CHC, CHC), jnp.int32),
            pltpu.VMEM((TW // CHC, CHC), jnp.int32),
            pltpu.VMEM((TW // CHC, K * CHC), jnp.int32),
            pltpu.VMEM((TW // CHC, CHC, K * 16), jnp.float32),
            pltpu.VMEM((2, K * CHC, D), jnp.float32),
            pltpu.VMEM((2, CHC, D), jnp.float32),
            pltpu.SemaphoreType.DMA,
            pltpu.SemaphoreType.DMA,
        ],
    )


@jax.jit
def kernel(x, gate, w1, w2, w3):
    pos0, pos1, wts, be = _router(x, gate)
    be_flat = be.reshape(128)

    p0d = pos0.reshape(NW, TW // CHD, CHD)
    p1d = pos1.reshape(NW, TW // CHD, CHD)
    p0c = pos0.reshape(NW, TW // CHC, CHC)
    p1c = pos1.reshape(NW, TW // CHC, CHC)
    wtc = wts.reshape(NW, TW // CHC, CHC, K * 16)

    xs = _dispatch()(x, p0d, p1d)
    ys = _gmm(be_flat, xs, w1, w3, w2)
    return _combine()(ys, p0c, p1c, wtc)
